# baseline (speedup 1.0000x reference)
import functools

import jax
import jax.numpy as jnp
from jax import lax
from jax.experimental import pallas as pl
from jax.experimental.pallas import tpu as pltpu
from jax.experimental.pallas import tpu_sc as plsc

_NB = 1024
_LT = 128
_XB = 4
_DEG = 32
_D = 50


def _tc1_body(qT_ref, entT_ref, BT_ref, Went_ref, bent_ref, c_ref, out_ref):
    entp = jnp.dot(Went_ref[...], entT_ref[...],
                   preferred_element_type=jnp.float32) + bent_ref[...]
    qB = jnp.dot(BT_ref[...], qT_ref[...], preferred_element_type=jnp.float32)
    out_ref[...] = jnp.sum(qB * entp, axis=0, keepdims=True) + c_ref[0, 0]


def _tc2_body(aggT_ref, Wproj_ref, gbias_ref, Wrank_ref, brank_ref, out_ref):
    blk = aggT_ref[...]
    agg = blk[0:_D, :] / (blk[63:64, :] + 1e-16)
    out_n = jnp.dot(Wproj_ref[...], agg,
                    preferred_element_type=jnp.float32) + gbias_ref[...]
    out_n = jnp.where(out_n > 0, out_n, jnp.exp(jnp.minimum(out_n, 0.0)) - 1.0)
    out_ref[...] = jnp.dot(Wrank_ref[...], out_n,
                           preferred_element_type=jnp.float32) + brank_ref[...]


def _tc_full_body(qT_ref, entT_ref, nbT_ref, BT_ref, v3_ref, Went_ref,
                  bent_ref, Wproj_ref, gbias_ref, Wrank_ref, brank_ref,
                  c_ref, out_ref):
    entp = jnp.dot(Went_ref[...], entT_ref[...],
                   preferred_element_type=jnp.float32) + bent_ref[...]
    qB = jnp.dot(BT_ref[...], qT_ref[...], preferred_element_type=jnp.float32)
    s_tgt = jnp.sum(qB * entp, axis=0, keepdims=True) + c_ref[0, 0]
    nb = nbT_ref[...]
    s_src = jnp.sum(nb * v3_ref[...], axis=0)
    s = s_src + s_tgt
    s = jnp.where(s > 0, s, 0.2 * s)
    e = jnp.exp(s)
    denom = jnp.sum(e, axis=0, keepdims=True) + 1e-16
    attn = e / denom
    agg = jnp.sum(nb * attn[None, :, :], axis=1)
    out_n = jnp.dot(Wproj_ref[...], agg,
                    preferred_element_type=jnp.float32) + gbias_ref[...]
    out_n = jnp.where(out_n > 0, out_n, jnp.exp(jnp.minimum(out_n, 0.0)) - 1.0)
    out_ref[...] = jnp.dot(Wrank_ref[...], out_n,
                           preferred_element_type=jnp.float32) + brank_ref[...]


def _sc_attention(nbT, stgt, vv):
    n_pad = 32 * _LT
    mesh = plsc.VectorSubcoreMesh(core_axis_name="c", subcore_axis_name="s")
    n_dc = _DEG // 8

    @functools.partial(
        pl.kernel, mesh=mesh,
        out_type=jax.ShapeDtypeStruct((64, n_pad), jnp.float32),
        scratch_types=[
            pltpu.VMEM((_D, 8, _LT), jnp.float32),
            pltpu.VMEM((_D, 8, _LT), jnp.float32),
            pltpu.VMEM((64, _LT), jnp.float32),
            pltpu.VMEM((_LT,), jnp.float32),
            pltpu.VMEM((64, 16), jnp.float32),
            pltpu.SemaphoreType.DMA,
            pltpu.SemaphoreType.DMA,
        ],
    )
    def sc_kernel(nbT_hbm, stgt_hbm, vv_hbm, out_hbm,
                  buf0, buf1, aggb, stgtb, vvb, sem0, sem1):
        wid = lax.axis_index("s") * 2 + lax.axis_index("c")
        pltpu.sync_copy(vv_hbm, vvb)
        bufs = (buf0, buf1)
        sems = (sem0, sem1)
        base = wid * _LT
        pltpu.sync_copy(stgt_hbm.at[pl.ds(base, _LT)], stgtb)

        def start(dc):
            return pltpu.async_copy(
                nbT_hbm.at[:, pl.ds(dc * 8, 8), pl.ds(base, _LT)],
                bufs[dc % 2], sems[dc % 2])

        cp = start(0)
        for dc in range(n_dc):
            nxt = start(dc + 1) if dc + 1 < n_dc else None
            cp.wait()
            buf = bufs[dc % 2]

            def j_body(j, _):
                js = j * 16
                st = stgtb[pl.ds(js, 16)]

                def k_a(step, accs):
                    k0 = step * 25
                    for u in range(25):
                        vk = vvb[k0 + u, pl.ds(0, 16)]
                        accs = tuple(
                            accs[d] + buf[k0 + u, d, pl.ds(js, 16)] * vk
                            for d in range(8))
                    return accs
                accs = lax.fori_loop(
                    0, _D // 25, k_a,
                    tuple(jnp.zeros((16,), jnp.float32) for _ in range(8)))

                es = []
                for d in range(8):
                    s = accs[d] + st
                    s = jnp.where(s > 0, s, 0.2 * s)
                    es.append(jnp.exp(s))
                esum = es[0]
                for d in range(1, 8):
                    esum = esum + es[d]
                if dc == 0:
                    aggb[63, pl.ds(js, 16)] = esum
                else:
                    aggb[63, pl.ds(js, 16)] = aggb[63, pl.ds(js, 16)] + esum

                def k_b(step, _):
                    k0 = step * 25
                    for u in range(25):
                        acc = es[0] * buf[k0 + u, 0, pl.ds(js, 16)]
                        for d in range(1, 8):
                            acc = acc + es[d] * buf[k0 + u, d, pl.ds(js, 16)]
                        if dc == 0:
                            aggb[k0 + u, pl.ds(js, 16)] = acc
                        else:
                            aggb[k0 + u, pl.ds(js, 16)] = (
                                aggb[k0 + u, pl.ds(js, 16)] + acc)
                    return 0
                lax.fori_loop(0, _D // 25, k_b, 0)
                return 0

            lax.fori_loop(0, _LT // 16, j_body, 0)
            cp = nxt
        pltpu.sync_copy(aggb, out_hbm.at[:, pl.ds(base, _LT)])

    return sc_kernel(nbT, stgt, vv)


@jax.jit
def kernel(query_emb, entity_emb, neighbors, W_ent, b_ent, A_bil, b_bil,
           W_proj, a_src, a_tgt, gat_bias, W_rank, b_rank):
    N, deg, D = neighbors.shape
    ent_in = entity_emb.shape[1]
    qT = jnp.transpose(query_emb, (2, 1, 0)).reshape(D, N)
    entT = entity_emb.T
    nbT = jnp.transpose(neighbors, (2, 1, 0))

    u = W_proj.T @ a_tgt
    v = W_proj.T @ a_src
    BT = jnp.einsum('k,kij->ji', u, A_bil)
    c = jnp.dot(b_bil, u).reshape(1, 1)

    n_sc = _XB * _NB
    const = lambda shape: pl.BlockSpec(shape, lambda i: (0,) * len(shape))

    stgt2d = pl.pallas_call(
        _tc1_body,
        grid=(_XB,),
        in_specs=[
            pl.BlockSpec((D, _NB), lambda i: (0, i)),
            pl.BlockSpec((ent_in, _NB), lambda i: (0, i)),
            const((D, D)),
            const((D, ent_in)),
            const((D, 1)),
            const((1, 1)),
        ],
        out_specs=pl.BlockSpec((1, _NB), lambda i: (0, i)),
        out_shape=jax.ShapeDtypeStruct((1, n_sc), jnp.float32),
    )(qT, entT, BT, W_ent, b_ent.reshape(D, 1), c)
    stgt = stgt2d.reshape(n_sc)

    vv = jnp.pad(jnp.broadcast_to(v.reshape(D, 1), (D, 16)),
                 ((0, 64 - D), (0, 0)))
    aggT = _sc_attention(nbT, stgt, vv)

    n_tc = pl.cdiv(N, _NB) * _NB - n_sc
    rank_tc = pl.pallas_call(
        _tc_full_body,
        grid=(n_tc // _NB,),
        in_specs=[
            pl.BlockSpec((D, _NB), lambda i: (0, i + _XB)),
            pl.BlockSpec((ent_in, _NB), lambda i: (0, i + _XB)),
            pl.BlockSpec((D, deg, _NB), lambda i: (0, 0, i + _XB)),
            const((D, D)),
            const((D, 1, 1)),
            const((D, ent_in)),
            const((D, 1)),
            const((D, D)),
            const((D, 1)),
            const((1, D)),
            const((1, 1)),
            const((1, 1)),
        ],
        out_specs=pl.BlockSpec((1, _NB), lambda i: (0, i)),
        out_shape=jax.ShapeDtypeStruct((1, n_tc), jnp.float32),
    )(
        qT, entT, nbT, BT, v.reshape(D, 1, 1), W_ent,
        b_ent.reshape(D, 1), W_proj, gat_bias.reshape(D, 1), W_rank,
        b_rank.reshape(1, 1), c,
    )

    brank2 = b_rank.reshape(1, 1) + 0.0 * rank_tc[:, :1]
    rank_sc = pl.pallas_call(
        _tc2_body,
        grid=(_XB,),
        in_specs=[
            pl.BlockSpec((64, _NB), lambda i: (0, i)),
            const((D, D)),
            const((D, 1)),
            const((1, D)),
            const((1, 1)),
        ],
        out_specs=pl.BlockSpec((1, _NB), lambda i: (0, i)),
        out_shape=jax.ShapeDtypeStruct((1, n_sc), jnp.float32),
    )(aggT, W_proj, gat_bias.reshape(D, 1), W_rank, brank2)

    rank = jnp.concatenate([rank_sc, rank_tc], axis=1)[:, :N]
    return rank.T


# speedup vs baseline: 1.2608x; 1.2608x over previous
import functools

import jax
import jax.numpy as jnp
from jax import lax
from jax.experimental import pallas as pl
from jax.experimental.pallas import tpu as pltpu
from jax.experimental.pallas import tpu_sc as plsc

_NB = 1024
_LT = 128
_XB = 4
_DEG = 32
_D = 50


def _tc1_body(qT_ref, entT_ref, BT_ref, Went_ref, bent_ref, c_ref, out_ref):
    entp = jnp.dot(Went_ref[...], entT_ref[...],
                   preferred_element_type=jnp.float32) + bent_ref[...]
    qB = jnp.dot(BT_ref[...], qT_ref[...], preferred_element_type=jnp.float32)
    out_ref[...] = jnp.sum(qB * entp, axis=0, keepdims=True) + c_ref[0, 0]


def _tc2_body(aggT_ref, Wproj_ref, gbias_ref, Wrank_ref, brank_ref, out_ref):
    blk = aggT_ref[...]
    agg = blk[0:_D, :] / (blk[63:64, :] + 1e-16)
    out_n = jnp.dot(Wproj_ref[...], agg,
                    preferred_element_type=jnp.float32) + gbias_ref[...]
    out_n = jnp.where(out_n > 0, out_n, jnp.exp(jnp.minimum(out_n, 0.0)) - 1.0)
    out_ref[...] = jnp.dot(Wrank_ref[...], out_n,
                           preferred_element_type=jnp.float32) + brank_ref[...]


def _tc_full_body(qT_ref, entT_ref, nbT_ref, BT_ref, v3_ref, Went_ref,
                  bent_ref, Wproj_ref, gbias_ref, Wrank_ref, brank_ref,
                  c_ref, out_ref):
    entp = jnp.dot(Went_ref[...], entT_ref[...],
                   preferred_element_type=jnp.float32) + bent_ref[...]
    qB = jnp.dot(BT_ref[...], qT_ref[...], preferred_element_type=jnp.float32)
    s_tgt = jnp.sum(qB * entp, axis=0, keepdims=True) + c_ref[0, 0]
    nb = nbT_ref[...]
    s_src = jnp.sum(nb * v3_ref[...], axis=0)
    s = s_src + s_tgt
    s = jnp.where(s > 0, s, 0.2 * s)
    e = jnp.exp(s)
    denom = jnp.sum(e, axis=0, keepdims=True) + 1e-16
    attn = e / denom
    agg = jnp.sum(nb * attn[None, :, :], axis=1)
    out_n = jnp.dot(Wproj_ref[...], agg,
                    preferred_element_type=jnp.float32) + gbias_ref[...]
    out_n = jnp.where(out_n > 0, out_n, jnp.exp(jnp.minimum(out_n, 0.0)) - 1.0)
    out_ref[...] = jnp.dot(Wrank_ref[...], out_n,
                           preferred_element_type=jnp.float32) + brank_ref[...]


def _sc_attention(nbT, stgt, vv):
    n_pad = 32 * _LT
    mesh = plsc.VectorSubcoreMesh(core_axis_name="c", subcore_axis_name="s")
    n_dc = _DEG // 8

    @functools.partial(
        pl.kernel, mesh=mesh,
        out_type=jax.ShapeDtypeStruct((64, n_pad), jnp.float32),
        scratch_types=[
            pltpu.VMEM((_D, 8, _LT), jnp.float32),
            pltpu.VMEM((_D, 8, _LT), jnp.float32),
            pltpu.VMEM((64, _LT), jnp.float32),
            pltpu.VMEM((_LT,), jnp.float32),
            pltpu.VMEM((64, 16), jnp.float32),
            pltpu.SemaphoreType.DMA,
            pltpu.SemaphoreType.DMA,
        ],
    )
    def sc_kernel(nbT_hbm, stgt_hbm, vv_hbm, out_hbm,
                  buf0, buf1, aggb, stgtb, vvb, sem0, sem1):
        wid = lax.axis_index("s") * 2 + lax.axis_index("c")
        pltpu.sync_copy(vv_hbm, vvb)
        bufs = (buf0, buf1)
        sems = (sem0, sem1)
        base = wid * _LT
        pltpu.sync_copy(stgt_hbm.at[pl.ds(base, _LT)], stgtb)

        def start(dc):
            return pltpu.async_copy(
                nbT_hbm.at[:, pl.ds(dc * 8, 8), pl.ds(base, _LT)],
                bufs[dc % 2], sems[dc % 2])

        cp = start(0)
        for dc in range(n_dc):
            nxt = start(dc + 1) if dc + 1 < n_dc else None
            cp.wait()
            buf = bufs[dc % 2]

            def j_body(j, _):
                js = j * 16
                st = stgtb[pl.ds(js, 16)]

                def k_a(step, accs):
                    k0 = step * 5
                    for u in range(5):
                        vk = vvb[k0 + u, pl.ds(0, 16)]
                        accs = tuple(
                            accs[d] + buf[k0 + u, d, pl.ds(js, 16)] * vk
                            for d in range(8))
                    return accs
                accs = lax.fori_loop(
                    0, _D // 5, k_a,
                    tuple(jnp.zeros((16,), jnp.float32) for _ in range(8)))

                es = []
                for d in range(8):
                    s = accs[d] + st
                    s = jnp.where(s > 0, s, 0.2 * s)
                    es.append(jnp.exp(s))
                esum = es[0]
                for d in range(1, 8):
                    esum = esum + es[d]
                if dc == 0:
                    aggb[63, pl.ds(js, 16)] = esum
                else:
                    aggb[63, pl.ds(js, 16)] = aggb[63, pl.ds(js, 16)] + esum

                def k_b(step, _):
                    k0 = step * 5
                    for u in range(5):
                        acc = es[0] * buf[k0 + u, 0, pl.ds(js, 16)]
                        for d in range(1, 8):
                            acc = acc + es[d] * buf[k0 + u, d, pl.ds(js, 16)]
                        if dc == 0:
                            aggb[k0 + u, pl.ds(js, 16)] = acc
                        else:
                            aggb[k0 + u, pl.ds(js, 16)] = (
                                aggb[k0 + u, pl.ds(js, 16)] + acc)
                    return 0
                lax.fori_loop(0, _D // 5, k_b, 0)
                return 0

            lax.fori_loop(0, _LT // 16, j_body, 0)
            cp = nxt
        pltpu.sync_copy(aggb, out_hbm.at[:, pl.ds(base, _LT)])

    return sc_kernel(nbT, stgt, vv)


@jax.jit
def kernel(query_emb, entity_emb, neighbors, W_ent, b_ent, A_bil, b_bil,
           W_proj, a_src, a_tgt, gat_bias, W_rank, b_rank):
    N, deg, D = neighbors.shape
    ent_in = entity_emb.shape[1]
    qT = jnp.transpose(query_emb, (2, 1, 0)).reshape(D, N)
    entT = entity_emb.T
    nbT = jnp.transpose(neighbors, (2, 1, 0))

    u = W_proj.T @ a_tgt
    v = W_proj.T @ a_src
    BT = jnp.einsum('k,kij->ji', u, A_bil)
    c = jnp.dot(b_bil, u).reshape(1, 1)

    n_sc = _XB * _NB
    const = lambda shape: pl.BlockSpec(shape, lambda i: (0,) * len(shape))

    stgt2d = pl.pallas_call(
        _tc1_body,
        grid=(_XB,),
        in_specs=[
            pl.BlockSpec((D, _NB), lambda i: (0, i)),
            pl.BlockSpec((ent_in, _NB), lambda i: (0, i)),
            const((D, D)),
            const((D, ent_in)),
            const((D, 1)),
            const((1, 1)),
        ],
        out_specs=pl.BlockSpec((1, _NB), lambda i: (0, i)),
        out_shape=jax.ShapeDtypeStruct((1, n_sc), jnp.float32),
    )(qT, entT, BT, W_ent, b_ent.reshape(D, 1), c)
    stgt = stgt2d.reshape(n_sc)

    vv = jnp.pad(jnp.broadcast_to(v.reshape(D, 1), (D, 16)),
                 ((0, 64 - D), (0, 0)))
    aggT = _sc_attention(nbT, stgt, vv)

    n_tc = pl.cdiv(N, _NB) * _NB - n_sc
    rank_tc = pl.pallas_call(
        _tc_full_body,
        grid=(n_tc // _NB,),
        in_specs=[
            pl.BlockSpec((D, _NB), lambda i: (0, i + _XB)),
            pl.BlockSpec((ent_in, _NB), lambda i: (0, i + _XB)),
            pl.BlockSpec((D, deg, _NB), lambda i: (0, 0, i + _XB)),
            const((D, D)),
            const((D, 1, 1)),
            const((D, ent_in)),
            const((D, 1)),
            const((D, D)),
            const((D, 1)),
            const((1, D)),
            const((1, 1)),
            const((1, 1)),
        ],
        out_specs=pl.BlockSpec((1, _NB), lambda i: (0, i)),
        out_shape=jax.ShapeDtypeStruct((1, n_tc), jnp.float32),
    )(
        qT, entT, nbT, BT, v.reshape(D, 1, 1), W_ent,
        b_ent.reshape(D, 1), W_proj, gat_bias.reshape(D, 1), W_rank,
        b_rank.reshape(1, 1), c,
    )

    brank2 = b_rank.reshape(1, 1) + 0.0 * rank_tc[:, :1]
    rank_sc = pl.pallas_call(
        _tc2_body,
        grid=(_XB,),
        in_specs=[
            pl.BlockSpec((64, _NB), lambda i: (0, i)),
            const((D, D)),
            const((D, 1)),
            const((1, D)),
            const((1, 1)),
        ],
        out_specs=pl.BlockSpec((1, _NB), lambda i: (0, i)),
        out_shape=jax.ShapeDtypeStruct((1, n_sc), jnp.float32),
    )(aggT, W_proj, gat_bias.reshape(D, 1), W_rank, brank2)

    rank = jnp.concatenate([rank_sc, rank_tc], axis=1)[:, :N]
    return rank.T


# speedup vs baseline: 1.2987x; 1.0300x over previous
import functools

import jax
import jax.numpy as jnp
from jax import lax
from jax.experimental import pallas as pl
from jax.experimental.pallas import tpu as pltpu
from jax.experimental.pallas import tpu_sc as plsc

_NB = 1024
_LT = 128
_XB = 4
_DEG = 32
_D = 50


def _tc1_body(qT_ref, entT_ref, BT_ref, Went_ref, bent_ref, c_ref, out_ref):
    entp = jnp.dot(Went_ref[...], entT_ref[...],
                   preferred_element_type=jnp.float32) + bent_ref[...]
    qB = jnp.dot(BT_ref[...], qT_ref[...], preferred_element_type=jnp.float32)
    out_ref[...] = jnp.sum(qB * entp, axis=0, keepdims=True) + c_ref[0, 0]


def _tc2_body(aggT_ref, Wproj_ref, gbias_ref, Wrank_ref, brank_ref, out_ref):
    blk = aggT_ref[...]
    agg = blk[0:_D, :] / (blk[63:64, :] + 1e-16)
    out_n = jnp.dot(Wproj_ref[...], agg,
                    preferred_element_type=jnp.float32) + gbias_ref[...]
    out_n = jnp.where(out_n > 0, out_n, jnp.exp(jnp.minimum(out_n, 0.0)) - 1.0)
    out_ref[...] = jnp.dot(Wrank_ref[...], out_n,
                           preferred_element_type=jnp.float32) + brank_ref[...]


def _tc_full_body(qT_ref, entT_ref, nbT_ref, BT_ref, v3_ref, Went_ref,
                  bent_ref, Wproj_ref, gbias_ref, Wrank_ref, brank_ref,
                  c_ref, out_ref):
    entp = jnp.dot(Went_ref[...], entT_ref[...],
                   preferred_element_type=jnp.float32) + bent_ref[...]
    qB = jnp.dot(BT_ref[...], qT_ref[...], preferred_element_type=jnp.float32)
    s_tgt = jnp.sum(qB * entp, axis=0, keepdims=True) + c_ref[0, 0]
    nb = nbT_ref[...]
    s_src = jnp.sum(nb * v3_ref[...], axis=0)
    s = s_src + s_tgt
    s = jnp.where(s > 0, s, 0.2 * s)
    e = jnp.exp(s)
    denom = jnp.sum(e, axis=0, keepdims=True) + 1e-16
    attn = e / denom
    agg = jnp.sum(nb * attn[None, :, :], axis=1)
    out_n = jnp.dot(Wproj_ref[...], agg,
                    preferred_element_type=jnp.float32) + gbias_ref[...]
    out_n = jnp.where(out_n > 0, out_n, jnp.exp(jnp.minimum(out_n, 0.0)) - 1.0)
    out_ref[...] = jnp.dot(Wrank_ref[...], out_n,
                           preferred_element_type=jnp.float32) + brank_ref[...]


def _sc_attention(nbT, stgt, vv):
    n_pad = 32 * _LT
    mesh = plsc.VectorSubcoreMesh(core_axis_name="c", subcore_axis_name="s")
    n_dc = _DEG // 8

    @functools.partial(
        pl.kernel, mesh=mesh,
        out_type=jax.ShapeDtypeStruct((64, n_pad), jnp.float32),
        scratch_types=[
            pltpu.VMEM((_D, 8, _LT), jnp.float32),
            pltpu.VMEM((_D, 8, _LT), jnp.float32),
            pltpu.VMEM((64, _LT), jnp.float32),
            pltpu.VMEM((_LT,), jnp.float32),
            pltpu.VMEM((64, 16), jnp.float32),
            pltpu.SemaphoreType.DMA,
            pltpu.SemaphoreType.DMA,
            pltpu.SemaphoreType.DMA,
        ],
    )
    def sc_kernel(nbT_hbm, stgt_hbm, vv_hbm, out_hbm,
                  buf0, buf1, aggb, stgtb, vvb, sem0, sem1, sem2):
        wid = lax.axis_index("s") * 2 + lax.axis_index("c")
        bufs = (buf0, buf1)
        sems = (sem0, sem1)
        base = wid * _LT
        cp_v = pltpu.async_copy(vv_hbm, vvb, sem2)
        cp_s = pltpu.async_copy(stgt_hbm.at[pl.ds(base, _LT)], stgtb, sem2)

        def start(dc):
            return pltpu.async_copy(
                nbT_hbm.at[:, pl.ds(dc * 8, 8), pl.ds(base, _LT)],
                bufs[dc % 2], sems[dc % 2])

        cp = start(0)
        cp_v.wait()
        cp_s.wait()
        for dc in range(n_dc):
            nxt = start(dc + 1) if dc + 1 < n_dc else None
            cp.wait()
            buf = bufs[dc % 2]

            def j_body(j, _):
                js = j * 16
                st = stgtb[pl.ds(js, 16)]

                def k_a(step, accs):
                    k0 = step * 5
                    for u in range(5):
                        vk = vvb[k0 + u, pl.ds(0, 16)]
                        accs = tuple(
                            accs[d] + buf[k0 + u, d, pl.ds(js, 16)] * vk
                            for d in range(8))
                    return accs
                accs = lax.fori_loop(
                    0, _D // 5, k_a,
                    tuple(jnp.zeros((16,), jnp.float32) for _ in range(8)))

                es = []
                for d in range(8):
                    s = accs[d] + st
                    s = jnp.where(s > 0, s, 0.2 * s)
                    es.append(jnp.exp(s))
                esum = es[0]
                for d in range(1, 8):
                    esum = esum + es[d]
                if dc == 0:
                    aggb[63, pl.ds(js, 16)] = esum
                else:
                    aggb[63, pl.ds(js, 16)] = aggb[63, pl.ds(js, 16)] + esum

                def k_b(step, _):
                    k0 = step * 5
                    for u in range(5):
                        acc = es[0] * buf[k0 + u, 0, pl.ds(js, 16)]
                        for d in range(1, 8):
                            acc = acc + es[d] * buf[k0 + u, d, pl.ds(js, 16)]
                        if dc == 0:
                            aggb[k0 + u, pl.ds(js, 16)] = acc
                        else:
                            aggb[k0 + u, pl.ds(js, 16)] = (
                                aggb[k0 + u, pl.ds(js, 16)] + acc)
                    return 0
                lax.fori_loop(0, _D // 5, k_b, 0)
                return 0

            lax.fori_loop(0, _LT // 16, j_body, 0)
            cp = nxt
        pltpu.sync_copy(aggb, out_hbm.at[:, pl.ds(base, _LT)])

    return sc_kernel(nbT, stgt, vv)


@jax.jit
def kernel(query_emb, entity_emb, neighbors, W_ent, b_ent, A_bil, b_bil,
           W_proj, a_src, a_tgt, gat_bias, W_rank, b_rank):
    N, deg, D = neighbors.shape
    ent_in = entity_emb.shape[1]
    qT = jnp.transpose(query_emb, (2, 1, 0)).reshape(D, N)
    entT = entity_emb.T
    nbT = jnp.transpose(neighbors, (2, 1, 0))

    u = W_proj.T @ a_tgt
    v = W_proj.T @ a_src
    BT = jnp.einsum('k,kij->ji', u, A_bil)
    c = jnp.dot(b_bil, u).reshape(1, 1)

    n_sc = _XB * _NB
    const = lambda shape: pl.BlockSpec(shape, lambda i: (0,) * len(shape))

    stgt2d = pl.pallas_call(
        _tc1_body,
        grid=(_XB,),
        in_specs=[
            pl.BlockSpec((D, _NB), lambda i: (0, i)),
            pl.BlockSpec((ent_in, _NB), lambda i: (0, i)),
            const((D, D)),
            const((D, ent_in)),
            const((D, 1)),
            const((1, 1)),
        ],
        out_specs=pl.BlockSpec((1, _NB), lambda i: (0, i)),
        out_shape=jax.ShapeDtypeStruct((1, n_sc), jnp.float32),
    )(qT, entT, BT, W_ent, b_ent.reshape(D, 1), c)
    stgt = stgt2d.reshape(n_sc)

    vv = jnp.pad(jnp.broadcast_to(v.reshape(D, 1), (D, 16)),
                 ((0, 64 - D), (0, 0)))
    aggT = _sc_attention(nbT, stgt, vv)

    n_tc = pl.cdiv(N, _NB) * _NB - n_sc
    rank_tc = pl.pallas_call(
        _tc_full_body,
        grid=(n_tc // _NB,),
        in_specs=[
            pl.BlockSpec((D, _NB), lambda i: (0, i + _XB)),
            pl.BlockSpec((ent_in, _NB), lambda i: (0, i + _XB)),
            pl.BlockSpec((D, deg, _NB), lambda i: (0, 0, i + _XB)),
            const((D, D)),
            const((D, 1, 1)),
            const((D, ent_in)),
            const((D, 1)),
            const((D, D)),
            const((D, 1)),
            const((1, D)),
            const((1, 1)),
            const((1, 1)),
        ],
        out_specs=pl.BlockSpec((1, _NB), lambda i: (0, i)),
        out_shape=jax.ShapeDtypeStruct((1, n_tc), jnp.float32),
    )(
        qT, entT, nbT, BT, v.reshape(D, 1, 1), W_ent,
        b_ent.reshape(D, 1), W_proj, gat_bias.reshape(D, 1), W_rank,
        b_rank.reshape(1, 1), c,
    )

    brank2 = b_rank.reshape(1, 1) + 0.0 * rank_tc[:, :1]
    rank_sc = pl.pallas_call(
        _tc2_body,
        grid=(_XB,),
        in_specs=[
            pl.BlockSpec((64, _NB), lambda i: (0, i)),
            const((D, D)),
            const((D, 1)),
            const((1, D)),
            const((1, 1)),
        ],
        out_specs=pl.BlockSpec((1, _NB), lambda i: (0, i)),
        out_shape=jax.ShapeDtypeStruct((1, n_sc), jnp.float32),
    )(aggT, W_proj, gat_bias.reshape(D, 1), W_rank, brank2)

    rank = jnp.concatenate([rank_sc, rank_tc], axis=1)[:, :N]
    return rank.T
